# baseline (speedup 1.0000x reference)
import jax
import jax.numpy as jnp
from jax import lax
from jax.experimental import pallas as pl
from jax.experimental.pallas import tpu as pltpu
from jax.experimental.pallas import tpu_sc as plsc

NC = 2
NS = 16
NW = NC * NS
CHUNK = 128
VB = 8192
QW = 2048


def _tc_body(in_ref, out_ref):
    x = in_ref[...]
    d = x.shape[0]
    row = lax.broadcasted_iota(jnp.int32, (d, d), 0)
    col = lax.broadcasted_iota(jnp.int32, (d, d), 1)
    eye = (row == col).astype(jnp.float32)
    parts = [
        lax.dot_general(
            x[:, q * QW : (q + 1) * QW],
            eye,
            dimension_numbers=(((0,), (0,)), ((), ())),
        )
        for q in range(VB // QW)
    ]
    out_ref[...] = jnp.concatenate(parts, axis=1)


def _relayout(table_t):
    D, V = table_t.shape
    grid = (V + VB - 1) // VB
    return pl.pallas_call(
        _tc_body,
        grid=(grid,),
        in_specs=[pl.BlockSpec((D, VB), lambda i: (0, i))],
        out_specs=pl.BlockSpec((QW, 128), lambda i: (i, 0)),
        out_shape=jax.ShapeDtypeStruct((grid * QW, 128), jnp.float32),
    )(table_t)


def _make_body(B, L, D, BPW, NCHUNK, NSTREAM):
    def body(idx_hbm, table_hbm, out_hbm, idx_v, acc_v, sem):
        wid = lax.axis_index("s") * NC + lax.axis_index("c")
        pltpu.sync_copy(idx_hbm.at[wid], idx_v)

        zeros = jnp.zeros((16,), jnp.float32)

        def zero_row(b, carry):
            for h in range(D // 16):
                acc_v[b, pl.ds(h * 16, 16)] = zeros
            return carry

        lax.fori_loop(0, BPW, zero_row, 0)

        def fire(r, carry):
            l = r // NCHUNK
            c = lax.rem(r, NCHUNK)
            pltpu.async_copy(
                table_hbm.at[idx_v.at[l, pl.ds(c * CHUNK, CHUNK)]],
                acc_v.at[pl.ds(c * CHUNK, CHUNK)],
                sem,
                add=True,
            )
            return carry

        lax.fori_loop(0, NSTREAM, fire, 0)

        def drain(r, carry):
            pltpu.make_async_copy(
                table_hbm.at[idx_v.at[0, pl.ds(0, CHUNK)]],
                acc_v.at[pl.ds(0, CHUNK)],
                sem,
            ).wait()
            return carry

        lax.fori_loop(0, NSTREAM, drain, 0)

        scale = jnp.float32(1.0 / L)

        def scale_row(b, carry):
            for h in range(D // 16):
                acc_v[b, pl.ds(h * 16, 16)] = acc_v[b, pl.ds(h * 16, 16)] * scale
            return carry

        lax.fori_loop(0, BPW, scale_row, 0)
        pltpu.sync_copy(acc_v, out_hbm.at[pl.ds(wid * BPW, BPW)])

    return body


def kernel(inputs, table):
    B, L = inputs.shape
    V, D = table.shape
    BPW = B // NW
    NCHUNK = BPW // CHUNK
    NSTREAM = L * NCHUNK

    v = inputs.astype(jnp.int32)
    j = v % VB
    pi = (v - j) + (v % QW) * 4 + j // QW
    idx = pi.reshape(NW, BPW, L).transpose(0, 2, 1)

    table_q = _relayout(table.T)
    table_rm = table_q.reshape(table_q.shape[0] * (128 // D), D)

    mesh = plsc.VectorSubcoreMesh(
        core_axis_name="c", subcore_axis_name="s", num_cores=NC, num_subcores=NS
    )
    f = pl.kernel(
        _make_body(B, L, D, BPW, NCHUNK, NSTREAM),
        out_type=jax.ShapeDtypeStruct((B, D), jnp.float32),
        mesh=mesh,
        scratch_types=[
            pltpu.VMEM((L, BPW), jnp.int32),
            pltpu.VMEM((BPW, D), jnp.float32),
            pltpu.SemaphoreType.DMA,
        ],
        compiler_params=pltpu.CompilerParams(
            use_tc_tiling_on_sc=False, needs_layout_passes=False
        ),
    )
    return f(idx, table_rm)


# speedup vs baseline: 1.0001x; 1.0001x over previous
import jax
import jax.numpy as jnp
from jax import lax
from jax.experimental import pallas as pl
from jax.experimental.pallas import tpu as pltpu
from jax.experimental.pallas import tpu_sc as plsc

NC = 2
NS = 16
NW = NC * NS
CHUNK = 128
VB = 8192
QW = 2048


def _tc_body(in_ref, out_ref):
    x = in_ref[...]
    d = x.shape[0]
    row = lax.broadcasted_iota(jnp.int32, (d, d), 0)
    col = lax.broadcasted_iota(jnp.int32, (d, d), 1)
    eye = (row == col).astype(jnp.float32)
    def tr(q):
        xq = x[:, q * QW : (q + 1) * QW]
        if q % 2 == 0:
            return jnp.transpose(xq)
        return lax.dot_general(
            xq, eye, dimension_numbers=(((0,), (0,)), ((), ()))
        )

    parts = [tr(q) for q in range(VB // QW)]
    out_ref[...] = jnp.concatenate(parts, axis=1)


def _relayout(table_t):
    D, V = table_t.shape
    grid = (V + VB - 1) // VB
    return pl.pallas_call(
        _tc_body,
        grid=(grid,),
        in_specs=[pl.BlockSpec((D, VB), lambda i: (0, i))],
        out_specs=pl.BlockSpec((QW, 128), lambda i: (i, 0)),
        out_shape=jax.ShapeDtypeStruct((grid * QW, 128), jnp.float32),
    )(table_t)


def _make_body(B, L, D, BPW, NCHUNK, NSTREAM):
    def body(idx_hbm, table_hbm, out_hbm, idx_v, acc_v, sem):
        wid = lax.axis_index("s") * NC + lax.axis_index("c")
        pltpu.sync_copy(idx_hbm.at[wid], idx_v)

        zeros = jnp.zeros((16,), jnp.float32)

        def zero_row(b, carry):
            for h in range(D // 16):
                acc_v[b, pl.ds(h * 16, 16)] = zeros
            return carry

        lax.fori_loop(0, BPW, zero_row, 0)

        def fire(r, carry):
            l = r // NCHUNK
            c = lax.rem(r, NCHUNK)
            pltpu.async_copy(
                table_hbm.at[idx_v.at[l, pl.ds(c * CHUNK, CHUNK)]],
                acc_v.at[pl.ds(c * CHUNK, CHUNK)],
                sem,
                add=True,
            )
            return carry

        lax.fori_loop(0, NSTREAM, fire, 0)

        def drain(r, carry):
            pltpu.make_async_copy(
                table_hbm.at[idx_v.at[0, pl.ds(0, CHUNK)]],
                acc_v.at[pl.ds(0, CHUNK)],
                sem,
            ).wait()
            return carry

        lax.fori_loop(0, NSTREAM, drain, 0)

        scale = jnp.float32(1.0 / L)

        def scale_row(b, carry):
            for h in range(D // 16):
                acc_v[b, pl.ds(h * 16, 16)] = acc_v[b, pl.ds(h * 16, 16)] * scale
            return carry

        lax.fori_loop(0, BPW, scale_row, 0)
        pltpu.sync_copy(acc_v, out_hbm.at[pl.ds(wid * BPW, BPW)])

    return body


def kernel(inputs, table):
    B, L = inputs.shape
    V, D = table.shape
    BPW = B // NW
    NCHUNK = BPW // CHUNK
    NSTREAM = L * NCHUNK

    v = inputs.astype(jnp.int32)
    j = v % VB
    pi = (v - j) + (v % QW) * 4 + j // QW
    idx = pi.reshape(NW, BPW, L).transpose(0, 2, 1)

    table_q = _relayout(table.T)
    table_rm = table_q.reshape(table_q.shape[0] * (128 // D), D)

    mesh = plsc.VectorSubcoreMesh(
        core_axis_name="c", subcore_axis_name="s", num_cores=NC, num_subcores=NS
    )
    f = pl.kernel(
        _make_body(B, L, D, BPW, NCHUNK, NSTREAM),
        out_type=jax.ShapeDtypeStruct((B, D), jnp.float32),
        mesh=mesh,
        scratch_types=[
            pltpu.VMEM((L, BPW), jnp.int32),
            pltpu.VMEM((BPW, D), jnp.float32),
            pltpu.SemaphoreType.DMA,
        ],
        compiler_params=pltpu.CompilerParams(
            use_tc_tiling_on_sc=False, needs_layout_passes=False
        ),
    )
    return f(idx, table_rm)


# speedup vs baseline: 1.5213x; 1.5212x over previous
import jax
import jax.numpy as jnp
from jax import lax
from jax.experimental import pallas as pl
from jax.experimental.pallas import tpu as pltpu
from jax.experimental.pallas import tpu_sc as plsc

NC = 2
NS = 16
NW = NC * NS
CHUNK = 128
VB = 8192
QW = 2048


def _tc_body(in_ref, out_ref):
    x = in_ref[...]
    z = jnp.concatenate(
        [x[:, q * QW : (q + 1) * QW] for q in range(VB // QW)], axis=0
    )
    out_ref[...] = jnp.transpose(z)


def _relayout(table_t):
    D, V = table_t.shape
    grid = (V + VB - 1) // VB
    return pl.pallas_call(
        _tc_body,
        grid=(grid,),
        in_specs=[pl.BlockSpec((D, VB), lambda i: (0, i))],
        out_specs=pl.BlockSpec((QW, 128), lambda i: (i, 0)),
        out_shape=jax.ShapeDtypeStruct((grid * QW, 128), jnp.float32),
    )(table_t)


def _make_body(B, L, D, BPW, NCHUNK, NSTREAM):
    def body(idx_hbm, table_hbm, out_hbm, idx_v, acc_v, sem):
        wid = lax.axis_index("s") * NC + lax.axis_index("c")
        pltpu.sync_copy(idx_hbm.at[wid], idx_v)

        zeros = jnp.zeros((16,), jnp.float32)

        def zero_row(b, carry):
            for h in range(D // 16):
                acc_v[b, pl.ds(h * 16, 16)] = zeros
            return carry

        lax.fori_loop(0, BPW, zero_row, 0)

        def fire(r, carry):
            l = r // NCHUNK
            c = lax.rem(r, NCHUNK)
            pltpu.async_copy(
                table_hbm.at[idx_v.at[l, pl.ds(c * CHUNK, CHUNK)]],
                acc_v.at[pl.ds(c * CHUNK, CHUNK)],
                sem,
                add=True,
            )
            return carry

        lax.fori_loop(0, NSTREAM, fire, 0)

        def drain(r, carry):
            pltpu.make_async_copy(
                table_hbm.at[idx_v.at[0, pl.ds(0, CHUNK)]],
                acc_v.at[pl.ds(0, CHUNK)],
                sem,
            ).wait()
            return carry

        lax.fori_loop(0, NSTREAM, drain, 0)

        scale = jnp.float32(1.0 / L)

        def scale_row(b, carry):
            for h in range(D // 16):
                acc_v[b, pl.ds(h * 16, 16)] = acc_v[b, pl.ds(h * 16, 16)] * scale
            return carry

        lax.fori_loop(0, BPW, scale_row, 0)
        pltpu.sync_copy(acc_v, out_hbm.at[pl.ds(wid * BPW, BPW)])

    return body


def kernel(inputs, table):
    B, L = inputs.shape
    V, D = table.shape
    BPW = B // NW
    NCHUNK = BPW // CHUNK
    NSTREAM = L * NCHUNK

    v = inputs.astype(jnp.int32)
    j = v % VB
    pi = (v - j) + (v % QW) * 4 + j // QW
    idx = pi.reshape(NW, BPW, L).transpose(0, 2, 1)

    table_q = _relayout(table.T)
    table_rm = table_q.reshape(table_q.shape[0] * (128 // D), D)

    mesh = plsc.VectorSubcoreMesh(
        core_axis_name="c", subcore_axis_name="s", num_cores=NC, num_subcores=NS
    )
    f = pl.kernel(
        _make_body(B, L, D, BPW, NCHUNK, NSTREAM),
        out_type=jax.ShapeDtypeStruct((B, D), jnp.float32),
        mesh=mesh,
        scratch_types=[
            pltpu.VMEM((L, BPW), jnp.int32),
            pltpu.VMEM((BPW, D), jnp.float32),
            pltpu.SemaphoreType.DMA,
        ],
        compiler_params=pltpu.CompilerParams(
            use_tc_tiling_on_sc=False, needs_layout_passes=False
        ),
    )
    return f(idx, table_rm)


# speedup vs baseline: 1.9865x; 1.3058x over previous
import jax
import jax.numpy as jnp
from jax import lax
from jax.experimental import pallas as pl
from jax.experimental.pallas import tpu as pltpu
from jax.experimental.pallas import tpu_sc as plsc

NC = 2
NS = 16
NW = NC * NS
CHUNK = 128
VB = 32768
QW = 8192


def _tc_body(in_ref, out_ref):
    x = in_ref[...]
    z = jnp.concatenate(
        [x[:, q * QW : (q + 1) * QW] for q in range(VB // QW)], axis=0
    )
    out_ref[...] = jnp.transpose(z)


def _relayout(table_t):
    D, V = table_t.shape
    grid = (V + VB - 1) // VB
    return pl.pallas_call(
        _tc_body,
        grid=(grid,),
        in_specs=[pl.BlockSpec((D, VB), lambda i: (0, i))],
        out_specs=pl.BlockSpec((QW, 128), lambda i: (i, 0)),
        out_shape=jax.ShapeDtypeStruct((grid * QW, 128), jnp.float32),
    )(table_t)


def _make_body(B, L, D, BPW, NCHUNK, NSTREAM):
    def body(idx_hbm, table_hbm, out_hbm, idx_v, acc_v, sem):
        wid = lax.axis_index("s") * NC + lax.axis_index("c")
        pltpu.sync_copy(idx_hbm.at[wid], idx_v)

        zeros = jnp.zeros((16,), jnp.float32)

        def zero_row(b, carry):
            for h in range(D // 16):
                acc_v[b, pl.ds(h * 16, 16)] = zeros
            return carry

        lax.fori_loop(0, BPW, zero_row, 0)

        def fire(r, carry):
            l = r // NCHUNK
            c = lax.rem(r, NCHUNK)
            pltpu.async_copy(
                table_hbm.at[idx_v.at[l, pl.ds(c * CHUNK, CHUNK)]],
                acc_v.at[pl.ds(c * CHUNK, CHUNK)],
                sem,
                add=True,
            )
            return carry

        lax.fori_loop(0, NSTREAM, fire, 0)

        def drain(r, carry):
            pltpu.make_async_copy(
                table_hbm.at[idx_v.at[0, pl.ds(0, CHUNK)]],
                acc_v.at[pl.ds(0, CHUNK)],
                sem,
            ).wait()
            return carry

        lax.fori_loop(0, NSTREAM, drain, 0)

        scale = jnp.float32(1.0 / L)

        def scale_row(b, carry):
            for h in range(D // 16):
                acc_v[b, pl.ds(h * 16, 16)] = acc_v[b, pl.ds(h * 16, 16)] * scale
            return carry

        lax.fori_loop(0, BPW, scale_row, 0)
        pltpu.sync_copy(acc_v, out_hbm.at[pl.ds(wid * BPW, BPW)])

    return body


def kernel(inputs, table):
    B, L = inputs.shape
    V, D = table.shape
    BPW = B // NW
    NCHUNK = BPW // CHUNK
    NSTREAM = L * NCHUNK

    v = inputs.astype(jnp.int32)
    j = v % VB
    pi = (v - j) + (v % QW) * 4 + j // QW
    idx = pi.reshape(NW, BPW, L).transpose(0, 2, 1)

    table_q = _relayout(table.T)
    table_rm = table_q.reshape(table_q.shape[0] * (128 // D), D)

    mesh = plsc.VectorSubcoreMesh(
        core_axis_name="c", subcore_axis_name="s", num_cores=NC, num_subcores=NS
    )
    f = pl.kernel(
        _make_body(B, L, D, BPW, NCHUNK, NSTREAM),
        out_type=jax.ShapeDtypeStruct((B, D), jnp.float32),
        mesh=mesh,
        scratch_types=[
            pltpu.VMEM((L, BPW), jnp.int32),
            pltpu.VMEM((BPW, D), jnp.float32),
            pltpu.SemaphoreType.DMA,
        ],
        compiler_params=pltpu.CompilerParams(
            use_tc_tiling_on_sc=False, needs_layout_passes=False
        ),
    )
    return f(idx, table_rm)


# speedup vs baseline: 1.9942x; 1.0038x over previous
import jax
import jax.numpy as jnp
from jax import lax
from jax.experimental import pallas as pl
from jax.experimental.pallas import tpu as pltpu
from jax.experimental.pallas import tpu_sc as plsc

NC = 2
NS = 16
NW = NC * NS
CHUNK = 128
VB = 65536
QW = 16384


def _tc_body(in_ref, out_ref):
    x = in_ref[...]
    z = jnp.concatenate(
        [x[:, q * QW : (q + 1) * QW] for q in range(VB // QW)], axis=0
    )
    out_ref[...] = jnp.transpose(z)


def _relayout(table_t):
    D, V = table_t.shape
    grid = (V + VB - 1) // VB
    return pl.pallas_call(
        _tc_body,
        grid=(grid,),
        in_specs=[pl.BlockSpec((D, VB), lambda i: (0, i))],
        out_specs=pl.BlockSpec((QW, 128), lambda i: (i, 0)),
        out_shape=jax.ShapeDtypeStruct((grid * QW, 128), jnp.float32),
    )(table_t)


def _make_body(B, L, D, BPW, NCHUNK, NSTREAM):
    def body(idx_hbm, table_hbm, out_hbm, idx_v, acc_v, sem):
        wid = lax.axis_index("s") * NC + lax.axis_index("c")
        pltpu.sync_copy(idx_hbm.at[wid], idx_v)

        zeros = jnp.zeros((16,), jnp.float32)

        def zero_row(b, carry):
            for h in range(D // 16):
                acc_v[b, pl.ds(h * 16, 16)] = zeros
            return carry

        lax.fori_loop(0, BPW, zero_row, 0)

        def fire(r, carry):
            l = r // NCHUNK
            c = lax.rem(r, NCHUNK)
            pltpu.async_copy(
                table_hbm.at[idx_v.at[l, pl.ds(c * CHUNK, CHUNK)]],
                acc_v.at[pl.ds(c * CHUNK, CHUNK)],
                sem,
                add=True,
            )
            return carry

        lax.fori_loop(0, NSTREAM, fire, 0)

        def drain(r, carry):
            pltpu.make_async_copy(
                table_hbm.at[idx_v.at[0, pl.ds(0, CHUNK)]],
                acc_v.at[pl.ds(0, CHUNK)],
                sem,
            ).wait()
            return carry

        lax.fori_loop(0, NSTREAM, drain, 0)

        scale = jnp.float32(1.0 / L)

        def scale_row(b, carry):
            for h in range(D // 16):
                acc_v[b, pl.ds(h * 16, 16)] = acc_v[b, pl.ds(h * 16, 16)] * scale
            return carry

        lax.fori_loop(0, BPW, scale_row, 0)
        pltpu.sync_copy(acc_v, out_hbm.at[pl.ds(wid * BPW, BPW)])

    return body


def kernel(inputs, table):
    B, L = inputs.shape
    V, D = table.shape
    BPW = B // NW
    NCHUNK = BPW // CHUNK
    NSTREAM = L * NCHUNK

    v = inputs.astype(jnp.int32)
    j = v % VB
    pi = (v - j) + (v % QW) * 4 + j // QW
    idx = pi.reshape(NW, BPW, L).transpose(0, 2, 1)

    table_q = _relayout(table.T)
    table_rm = table_q.reshape(table_q.shape[0] * (128 // D), D)

    mesh = plsc.VectorSubcoreMesh(
        core_axis_name="c", subcore_axis_name="s", num_cores=NC, num_subcores=NS
    )
    f = pl.kernel(
        _make_body(B, L, D, BPW, NCHUNK, NSTREAM),
        out_type=jax.ShapeDtypeStruct((B, D), jnp.float32),
        mesh=mesh,
        scratch_types=[
            pltpu.VMEM((L, BPW), jnp.int32),
            pltpu.VMEM((BPW, D), jnp.float32),
            pltpu.SemaphoreType.DMA,
        ],
        compiler_params=pltpu.CompilerParams(
            use_tc_tiling_on_sc=False, needs_layout_passes=False
        ),
    )
    return f(idx, table_rm)


# speedup vs baseline: 2.0339x; 1.0199x over previous
import jax
import jax.numpy as jnp
from jax import lax
from jax.experimental import pallas as pl
from jax.experimental.pallas import tpu as pltpu
from jax.experimental.pallas import tpu_sc as plsc

NC = 2
NS = 16
NW = NC * NS
CHUNK = 128
VB = 65536
QW = 16384


def _tc_body(in_ref, out_ref):
    x = in_ref[...]
    z = jnp.concatenate(
        [x[:, q * QW : (q + 1) * QW] for q in range(VB // QW)], axis=0
    )
    out_ref[...] = jnp.transpose(z)


def _relayout(table_t):
    D, V = table_t.shape
    grid = (V + VB - 1) // VB
    return pl.pallas_call(
        _tc_body,
        grid=(grid,),
        in_specs=[pl.BlockSpec((D, VB), lambda i: (0, i))],
        out_specs=pl.BlockSpec((QW, 128), lambda i: (i, 0)),
        out_shape=jax.ShapeDtypeStruct((grid * QW, 128), jnp.float32),
    )(table_t)


def _make_body(B, L, D, BPW, NCHUNK, NSTREAM):
    def body(idx_hbm, table_hbm, out_hbm, idx_v, acc_v, sem):
        wid = lax.axis_index("s") * NC + lax.axis_index("c")
        pltpu.sync_copy(idx_hbm.at[:, wid], idx_v)

        zeros = jnp.zeros((16,), jnp.float32)

        def zero_row(b, carry):
            for h in range(D // 16):
                acc_v[b, pl.ds(h * 16, 16)] = zeros
            return carry

        lax.fori_loop(0, BPW, zero_row, 0)

        def fire(r, carry):
            l = r // NCHUNK
            c = lax.rem(r, NCHUNK)
            pltpu.async_copy(
                table_hbm.at[idx_v.at[l, pl.ds(c * CHUNK, CHUNK)]],
                acc_v.at[pl.ds(c * CHUNK, CHUNK)],
                sem,
                add=True,
            )
            return carry

        lax.fori_loop(0, NSTREAM, fire, 0)

        def drain(r, carry):
            pltpu.make_async_copy(
                table_hbm.at[idx_v.at[0, pl.ds(0, CHUNK)]],
                acc_v.at[pl.ds(0, CHUNK)],
                sem,
            ).wait()
            return carry

        lax.fori_loop(0, NSTREAM, drain, 0)

        scale = jnp.float32(1.0 / L)

        def scale_row(b, carry):
            for h in range(D // 16):
                acc_v[b, pl.ds(h * 16, 16)] = acc_v[b, pl.ds(h * 16, 16)] * scale
            return carry

        lax.fori_loop(0, BPW, scale_row, 0)
        pltpu.sync_copy(acc_v, out_hbm.at[pl.ds(wid * BPW, BPW)])

    return body


def kernel(inputs, table):
    B, L = inputs.shape
    V, D = table.shape
    BPW = B // NW
    NCHUNK = BPW // CHUNK
    NSTREAM = L * NCHUNK

    v = inputs.astype(jnp.int32)
    j = v % VB
    pi = (v - j) + (v % QW) * 4 + j // QW
    idx = pi.T.reshape(L, NW, BPW)

    table_q = _relayout(table.T)
    table_rm = table_q.reshape(table_q.shape[0] * (128 // D), D)

    mesh = plsc.VectorSubcoreMesh(
        core_axis_name="c", subcore_axis_name="s", num_cores=NC, num_subcores=NS
    )
    f = pl.kernel(
        _make_body(B, L, D, BPW, NCHUNK, NSTREAM),
        out_type=jax.ShapeDtypeStruct((B, D), jnp.float32),
        mesh=mesh,
        scratch_types=[
            pltpu.VMEM((L, BPW), jnp.int32),
            pltpu.VMEM((BPW, D), jnp.float32),
            pltpu.SemaphoreType.DMA,
        ],
        compiler_params=pltpu.CompilerParams(
            use_tc_tiling_on_sc=False, needs_layout_passes=False
        ),
    )
    return f(idx, table_rm)
